# baseline (speedup 1.0000x reference)
import functools

import jax
import jax.numpy as jnp
from jax import lax
from jax.experimental import pallas as pl
from jax.experimental.pallas import tpu as pltpu
from jax.experimental.pallas import tpu_sc as plsc

N_NODES = 10000
N_EDGES = 320000
D = 128

NC = 2
NS = 16
NW = NC * NS

CHUNK = 112
NROWS = 2880
E_PAD = NROWS * CHUNK
RPT0 = 156
RPT1 = (NROWS - NS * RPT0) // NS
N_PAD = 10112
NODES_PER_TILE = N_PAD // NS
NBUF = 3
NI = 6
GROUPS = CHUNK // 16


def _make_sc_spmm():
    mesh = plsc.VectorSubcoreMesh(core_axis_name="c", subcore_axis_name="s")

    @functools.partial(
        pl.kernel,
        out_type=jax.ShapeDtypeStruct((NC, N_PAD, D), jnp.float32),
        mesh=mesh,
        scratch_types=[
            pltpu.VMEM((NI, CHUNK), jnp.int32),
            pltpu.VMEM((NI, CHUNK), jnp.int32),
            pltpu.VMEM((NI, CHUNK), jnp.float32),
            [pltpu.VMEM((CHUNK, D), jnp.float32)] * NBUF,
            pltpu.VMEM_SHARED((N_PAD, D), jnp.float32),
            [pltpu.SemaphoreType.DMA] * NI,
            [pltpu.SemaphoreType.DMA] * NBUF,
            [pltpu.SemaphoreType.DMA] * NBUF,
        ],
    )
    def sc_spmm(x_hbm, src_hbm, dst_hbm, vals_hbm, out_hbm,
                src_s, dst_s, val_s, rows, acc, isem, gsem, ssem):
        cid = lax.axis_index("c")
        sid = lax.axis_index("s")
        rpt = jnp.where(cid == 0, RPT0, RPT1)
        row0 = jnp.where(cid == 0, sid * RPT0, NS * RPT0 + sid * RPT1)

        zeros16 = jnp.zeros((16,), jnp.float32)
        dn = lax.GatherDimensionNumbers(
            offset_dims=(), collapsed_slice_dims=(0,), start_index_map=(0,))

        def zrow(i, _):
            for d in range(D // 16):
                rows[0][i, pl.ds(d * 16, 16)] = zeros16
            return 0

        lax.fori_loop(0, CHUNK, zrow, 0)
        nbase = sid * NODES_PER_TILE
        off = 0
        while off < NODES_PER_TILE:
            n = min(CHUNK, NODES_PER_TILE - off)
            pltpu.sync_copy(rows[0].at[pl.ds(0, n)],
                            acc.at[pl.ds(nbase + off, n)])
            off += n
        plsc.subcore_barrier()

        def idx_load(c, q):
            pltpu.async_copy(src_hbm.at[c], src_s.at[q], isem[q])
            pltpu.async_copy(dst_hbm.at[c], dst_s.at[q], isem[q])
            pltpu.async_copy(vals_hbm.at[c], val_s.at[q], isem[q])

        def idx_wait(c, q):
            for ref, slot in ((src_hbm, src_s), (dst_hbm, dst_s),
                              (vals_hbm, val_s)):
                pltpu.make_async_copy(ref.at[c], slot.at[q], isem[q]).wait()

        def gather_start(b, q):
            pltpu.async_copy(x_hbm.at[src_s.at[q]], rows[b], gsem[b])

        def gather_wait(b, q):
            pltpu.make_async_copy(x_hbm.at[src_s.at[q]], rows[b],
                                  gsem[b]).wait()

        def scatter_start(b, q):
            pltpu.async_copy(rows[b], acc.at[dst_s.at[q]], ssem[b], add=True)

        def scatter_wait(b, q):
            pltpu.make_async_copy(rows[b], acc.at[dst_s.at[q]],
                                  ssem[b]).wait()

        def scale(b, q):
            def group_body(g, _):
                v = val_s[q, pl.ds(g * 16, 16)]
                for j in range(16):
                    s = lax.gather(v, jnp.full((16, 1), j, jnp.int32), dn,
                                   (1,),
                                   mode=lax.GatherScatterMode.PROMISE_IN_BOUNDS)
                    e = g * 16 + j
                    for d in range(D // 16):
                        seg = rows[b][e, pl.ds(d * 16, 16)]
                        rows[b][e, pl.ds(d * 16, 16)] = seg * s
                return 0

            lax.fori_loop(0, GROUPS, group_body, 0)

        for q in range(4):
            idx_load(row0 + q, q)
        idx_wait(row0 + 0, 0)
        gather_start(0, 0)
        idx_wait(row0 + 1, 1)
        gather_start(1, 1)

        def step(i_dyn, i_mod6, do_d, do_e, do_fg):
            b = i_mod6 % NBUF
            q = i_mod6 % NI
            gather_wait(b, q)
            scale(b, q)
            scatter_start(b, q)
            if do_d:
                b1 = (i_mod6 + NBUF - 1) % NBUF
                q1 = (i_mod6 + NI - 1) % NI
                scatter_wait(b1, q1)
            if do_e:
                idx_load(i_dyn + 4, (i_mod6 + 4) % NI)
            if do_fg:
                q2 = (i_mod6 + 2) % NI
                b2 = (i_mod6 + 2) % NBUF
                idx_wait(i_dyn + 2, q2)
                gather_start(b2, q2)

        step(row0 + 0, 0, False, True, True)
        step(row0 + 1, 1, True, True, True)

        def super_body(k, _):
            base = row0 + 2 + 6 * k
            for j in range(6):
                step(base + j, 2 + j, True, True, True)
            return 0

        lax.fori_loop(0, rpt // 6 - 1, super_body, 0)

        tbase = row0 + rpt - 4
        step(tbase + 0, 2, True, False, True)
        step(tbase + 1, 3, True, False, True)
        step(tbase + 2, 4, True, False, False)
        step(tbase + 3, 5, True, False, False)
        scatter_wait(5 % NBUF, 5 % NI)

        plsc.subcore_barrier()

        off = 0
        while off < NODES_PER_TILE:
            n = min(CHUNK, NODES_PER_TILE - off)
            nrow = nbase + off
            pltpu.sync_copy(acc.at[pl.ds(nrow, n)], rows[0].at[pl.ds(0, n)])
            pltpu.sync_copy(rows[0].at[pl.ds(0, n)],
                            out_hbm.at[cid, pl.ds(nrow, n)])
            off += n

    return sc_spmm


_sc_spmm = _make_sc_spmm()


def _finish_body(p0_ref, p1_ref, x_ref, a_ref, x0_ref, o_ref):
    ax = p0_ref[0] + p1_ref[0]
    alph = jax.nn.sigmoid(a_ref[...])
    o_ref[...] = alph * (0.5 * (ax - x_ref[...])) + x0_ref[...]


def _finish(partial, x, alpha, x0):
    blk = 1000
    grid = N_NODES // blk
    return pl.pallas_call(
        _finish_body,
        grid=(grid,),
        in_specs=[
            pl.BlockSpec((1, blk, D), lambda i: (0, i, 0)),
            pl.BlockSpec((1, blk, D), lambda i: (1, i, 0)),
            pl.BlockSpec((blk, D), lambda i: (i, 0)),
            pl.BlockSpec((blk, 1), lambda i: (i, 0)),
            pl.BlockSpec((blk, D), lambda i: (i, 0)),
        ],
        out_specs=pl.BlockSpec((blk, D), lambda i: (i, 0)),
        out_shape=jax.ShapeDtypeStruct((N_NODES, D), jnp.float32),
    )(partial, partial, x, alpha, x0)


@jax.jit
def kernel(t, x, edge_index, adj_values, alpha_train, x0):
    del t
    src = edge_index[1].astype(jnp.int32)
    dst = edge_index[0].astype(jnp.int32)
    vals = adj_values.astype(jnp.float32)

    pad = E_PAD - N_EDGES
    src = jnp.pad(src, (0, pad)).reshape(-1, CHUNK)
    dst = jnp.pad(dst, (0, pad)).reshape(-1, CHUNK)
    vals = jnp.pad(vals, (0, pad)).reshape(-1, CHUNK)

    partial = _sc_spmm(x, src, dst, vals)
    return _finish(partial, x, alpha_train.reshape(-1, 1), x0)


# speedup vs baseline: 1.0585x; 1.0585x over previous
import functools

import jax
import jax.numpy as jnp
from jax import lax
from jax.experimental import pallas as pl
from jax.experimental.pallas import tpu as pltpu
from jax.experimental.pallas import tpu_sc as plsc

N_NODES = 10000
N_EDGES = 320000
D = 128

NC = 2
NS = 16
NW = NC * NS

CHUNK = 112
NROWS = 2880
E_PAD = NROWS * CHUNK
RPT0 = 144
RPT1 = (NROWS - NS * RPT0) // NS
N_PAD = 10112
NODES_PER_TILE = N_PAD // NS
NBUF = 3
NI = 6
GROUPS = CHUNK // 16


def _make_sc_spmm():
    mesh = plsc.VectorSubcoreMesh(core_axis_name="c", subcore_axis_name="s")

    @functools.partial(
        pl.kernel,
        out_type=jax.ShapeDtypeStruct((NC, N_PAD, D), jnp.float32),
        mesh=mesh,
        scratch_types=[
            pltpu.VMEM((NI, CHUNK), jnp.int32),
            pltpu.VMEM((NI, CHUNK), jnp.int32),
            pltpu.VMEM((NI, CHUNK), jnp.float32),
            [pltpu.VMEM((CHUNK, D), jnp.float32)] * NBUF,
            pltpu.VMEM_SHARED((N_PAD, D), jnp.float32),
            [pltpu.SemaphoreType.DMA] * NI,
            [pltpu.SemaphoreType.DMA] * NBUF,
            [pltpu.SemaphoreType.DMA] * NBUF,
        ],
    )
    def sc_spmm(x_hbm, src_hbm, dst_hbm, vals_hbm, out_hbm,
                src_s, dst_s, val_s, rows, acc, isem, gsem, ssem):
        cid = lax.axis_index("c")
        sid = lax.axis_index("s")
        rpt = jnp.where(cid == 0, RPT0, RPT1)
        row0 = jnp.where(cid == 0, sid * RPT0, NS * RPT0 + sid * RPT1)

        zeros16 = jnp.zeros((16,), jnp.float32)
        dn = lax.GatherDimensionNumbers(
            offset_dims=(), collapsed_slice_dims=(0,), start_index_map=(0,))

        def zrow(i, _):
            for d in range(D // 16):
                rows[0][i, pl.ds(d * 16, 16)] = zeros16
            return 0

        lax.fori_loop(0, CHUNK, zrow, 0)
        nbase = sid * NODES_PER_TILE
        off = 0
        while off < NODES_PER_TILE:
            n = min(CHUNK, NODES_PER_TILE - off)
            pltpu.sync_copy(rows[0].at[pl.ds(0, n)],
                            acc.at[pl.ds(nbase + off, n)])
            off += n
        plsc.subcore_barrier()

        def idx_load(c, q):
            pltpu.async_copy(src_hbm.at[c], src_s.at[q], isem[q])
            pltpu.async_copy(dst_hbm.at[c], dst_s.at[q], isem[q])
            pltpu.async_copy(vals_hbm.at[c], val_s.at[q], isem[q])

        def idx_wait(c, q):
            for ref, slot in ((src_hbm, src_s), (dst_hbm, dst_s),
                              (vals_hbm, val_s)):
                pltpu.make_async_copy(ref.at[c], slot.at[q], isem[q]).wait()

        def gather_start(b, q):
            pltpu.async_copy(x_hbm.at[src_s.at[q]], rows[b], gsem[b])

        def gather_wait(b, q):
            pltpu.make_async_copy(x_hbm.at[src_s.at[q]], rows[b],
                                  gsem[b]).wait()

        def scatter_start(b, q):
            pltpu.async_copy(rows[b], acc.at[dst_s.at[q]], ssem[b], add=True)

        def scatter_wait(b, q):
            pltpu.make_async_copy(rows[b], acc.at[dst_s.at[q]],
                                  ssem[b]).wait()

        def scale(b, q):
            def group_body(g, _):
                v = val_s[q, pl.ds(g * 16, 16)]
                for j in range(16):
                    s = lax.gather(v, jnp.full((16, 1), j, jnp.int32), dn,
                                   (1,),
                                   mode=lax.GatherScatterMode.PROMISE_IN_BOUNDS)
                    e = g * 16 + j
                    for d in range(D // 16):
                        seg = rows[b][e, pl.ds(d * 16, 16)]
                        rows[b][e, pl.ds(d * 16, 16)] = seg * s
                return 0

            lax.fori_loop(0, GROUPS, group_body, 0)

        for q in range(4):
            idx_load(row0 + q, q)
        idx_wait(row0 + 0, 0)
        gather_start(0, 0)
        idx_wait(row0 + 1, 1)
        gather_start(1, 1)

        def step(i_dyn, i_mod6, do_d, do_e, do_fg):
            b = i_mod6 % NBUF
            q = i_mod6 % NI
            gather_wait(b, q)
            scale(b, q)
            scatter_start(b, q)
            if do_d:
                b1 = (i_mod6 + NBUF - 1) % NBUF
                q1 = (i_mod6 + NI - 1) % NI
                scatter_wait(b1, q1)
            if do_e:
                idx_load(i_dyn + 4, (i_mod6 + 4) % NI)
            if do_fg:
                q2 = (i_mod6 + 2) % NI
                b2 = (i_mod6 + 2) % NBUF
                idx_wait(i_dyn + 2, q2)
                gather_start(b2, q2)

        step(row0 + 0, 0, False, True, True)
        step(row0 + 1, 1, True, True, True)

        def super_body(k, _):
            base = row0 + 2 + 6 * k
            for j in range(6):
                step(base + j, 2 + j, True, True, True)
            return 0

        lax.fori_loop(0, rpt // 6 - 1, super_body, 0)

        tbase = row0 + rpt - 4
        step(tbase + 0, 2, True, False, True)
        step(tbase + 1, 3, True, False, True)
        step(tbase + 2, 4, True, False, False)
        step(tbase + 3, 5, True, False, False)
        scatter_wait(5 % NBUF, 5 % NI)

        plsc.subcore_barrier()

        off = 0
        while off < NODES_PER_TILE:
            n = min(CHUNK, NODES_PER_TILE - off)
            nrow = nbase + off
            pltpu.sync_copy(acc.at[pl.ds(nrow, n)], rows[0].at[pl.ds(0, n)])
            pltpu.sync_copy(rows[0].at[pl.ds(0, n)],
                            out_hbm.at[cid, pl.ds(nrow, n)])
            off += n

    return sc_spmm


_sc_spmm = _make_sc_spmm()


def _finish_body(p0_ref, p1_ref, x_ref, a_ref, x0_ref, o_ref):
    ax = p0_ref[0] + p1_ref[0]
    alph = jax.nn.sigmoid(a_ref[...])
    o_ref[...] = alph * (0.5 * (ax - x_ref[...])) + x0_ref[...]


def _finish(partial, x, alpha, x0):
    blk = 1000
    grid = N_NODES // blk
    return pl.pallas_call(
        _finish_body,
        grid=(grid,),
        in_specs=[
            pl.BlockSpec((1, blk, D), lambda i: (0, i, 0)),
            pl.BlockSpec((1, blk, D), lambda i: (1, i, 0)),
            pl.BlockSpec((blk, D), lambda i: (i, 0)),
            pl.BlockSpec((blk, 1), lambda i: (i, 0)),
            pl.BlockSpec((blk, D), lambda i: (i, 0)),
        ],
        out_specs=pl.BlockSpec((blk, D), lambda i: (i, 0)),
        out_shape=jax.ShapeDtypeStruct((N_NODES, D), jnp.float32),
    )(partial, partial, x, alpha, x0)


@jax.jit
def kernel(t, x, edge_index, adj_values, alpha_train, x0):
    del t
    src = edge_index[1].astype(jnp.int32)
    dst = edge_index[0].astype(jnp.int32)
    vals = adj_values.astype(jnp.float32)

    pad = E_PAD - N_EDGES
    src = jnp.pad(src, (0, pad)).reshape(-1, CHUNK)
    dst = jnp.pad(dst, (0, pad)).reshape(-1, CHUNK)
    vals = jnp.pad(vals, (0, pad)).reshape(-1, CHUNK)

    partial = _sc_spmm(x, src, dst, vals)
    return _finish(partial, x, alpha_train.reshape(-1, 1), x0)


# speedup vs baseline: 1.0684x; 1.0094x over previous
import functools

import jax
import jax.numpy as jnp
from jax import lax
from jax.experimental import pallas as pl
from jax.experimental.pallas import tpu as pltpu
from jax.experimental.pallas import tpu_sc as plsc

N_NODES = 10000
N_EDGES = 320000
D = 128

NC = 2
NS = 16
NW = NC * NS

CHUNK = 112
NROWS = 2880
E_PAD = NROWS * CHUNK
TAIL_CHUNK0 = N_EDGES // CHUNK
TAIL_CHUNK0 = (N_EDGES // CHUNK // 8) * 8
TAIL_LEN = (NROWS - TAIL_CHUNK0) * CHUNK
RPT0 = 138
RPT1 = (NROWS - NS * RPT0) // NS
N_PAD = 10112
NODES_PER_TILE = N_PAD // NS
NBUF = 3
NI = 6
GROUPS = CHUNK // 16


def _make_sc_spmm():
    mesh = plsc.VectorSubcoreMesh(core_axis_name="c", subcore_axis_name="s")

    @functools.partial(
        pl.kernel,
        out_type=jax.ShapeDtypeStruct((NC, N_PAD, D), jnp.float32),
        mesh=mesh,
        scratch_types=[
            pltpu.VMEM((NI, CHUNK), jnp.int32),
            pltpu.VMEM((NI, CHUNK), jnp.int32),
            pltpu.VMEM((NI, CHUNK), jnp.float32),
            [pltpu.VMEM((CHUNK, D), jnp.float32)] * NBUF,
            pltpu.VMEM_SHARED((N_PAD, D), jnp.float32),
            [pltpu.SemaphoreType.DMA] * NI,
            [pltpu.SemaphoreType.DMA] * NBUF,
            [pltpu.SemaphoreType.DMA] * NBUF,
        ],
    )
    def sc_spmm(x_hbm, src_hbm, dst_hbm, vals_hbm,
                tsrc_hbm, tdst_hbm, tvals_hbm, out_hbm,
                src_s, dst_s, val_s, rows, acc, isem, gsem, ssem):
        cid = lax.axis_index("c")
        sid = lax.axis_index("s")
        rpt = jnp.where(cid == 0, RPT0, RPT1)
        row0 = jnp.where(cid == 0, sid * RPT0, NS * RPT0 + sid * RPT1)

        zeros16 = jnp.zeros((16,), jnp.float32)
        dn = lax.GatherDimensionNumbers(
            offset_dims=(), collapsed_slice_dims=(0,), start_index_map=(0,))

        def zrow(i, _):
            for d in range(D // 16):
                rows[0][i, pl.ds(d * 16, 16)] = zeros16
            return 0

        lax.fori_loop(0, CHUNK, zrow, 0)
        nbase = sid * NODES_PER_TILE
        off = 0
        while off < NODES_PER_TILE:
            n = min(CHUNK, NODES_PER_TILE - off)
            pltpu.sync_copy(rows[0].at[pl.ds(0, n)],
                            acc.at[pl.ds(nbase + off, n)])
            off += n
        plsc.subcore_barrier()

        def idx_load(c, q):
            @pl.when(c < TAIL_CHUNK0)
            def _():
                off = c * CHUNK
                pltpu.async_copy(src_hbm.at[pl.ds(off, CHUNK)],
                                 src_s.at[q], isem[q])
                pltpu.async_copy(dst_hbm.at[pl.ds(off, CHUNK)],
                                 dst_s.at[q], isem[q])
                pltpu.async_copy(vals_hbm.at[pl.ds(off, CHUNK)],
                                 val_s.at[q], isem[q])

            @pl.when(c >= TAIL_CHUNK0)
            def _():
                off = (c - TAIL_CHUNK0) * CHUNK
                pltpu.async_copy(tsrc_hbm.at[pl.ds(off, CHUNK)],
                                 src_s.at[q], isem[q])
                pltpu.async_copy(tdst_hbm.at[pl.ds(off, CHUNK)],
                                 dst_s.at[q], isem[q])
                pltpu.async_copy(tvals_hbm.at[pl.ds(off, CHUNK)],
                                 val_s.at[q], isem[q])

        def idx_wait(c, q):
            for ref, slot in ((src_hbm, src_s), (dst_hbm, dst_s),
                              (vals_hbm, val_s)):
                pltpu.make_async_copy(ref.at[pl.ds(0, CHUNK)],
                                      slot.at[q], isem[q]).wait()

        def gather_start(b, q):
            pltpu.async_copy(x_hbm.at[src_s.at[q]], rows[b], gsem[b])

        def gather_wait(b, q):
            pltpu.make_async_copy(x_hbm.at[src_s.at[q]], rows[b],
                                  gsem[b]).wait()

        def scatter_start(b, q):
            pltpu.async_copy(rows[b], acc.at[dst_s.at[q]], ssem[b], add=True)

        def scatter_wait(b, q):
            pltpu.make_async_copy(rows[b], acc.at[dst_s.at[q]],
                                  ssem[b]).wait()

        def scale(b, q):
            def group_body(g, _):
                v = val_s[q, pl.ds(g * 16, 16)]
                for j in range(16):
                    s = lax.gather(v, jnp.full((16, 1), j, jnp.int32), dn,
                                   (1,),
                                   mode=lax.GatherScatterMode.PROMISE_IN_BOUNDS)
                    e = g * 16 + j
                    for d in range(D // 16):
                        seg = rows[b][e, pl.ds(d * 16, 16)]
                        rows[b][e, pl.ds(d * 16, 16)] = seg * s
                return 0

            lax.fori_loop(0, GROUPS, group_body, 0)

        for q in range(4):
            idx_load(row0 + q, q)
        idx_wait(row0 + 0, 0)
        gather_start(0, 0)
        idx_wait(row0 + 1, 1)
        gather_start(1, 1)

        def step(i_dyn, i_mod6, do_d, do_e, do_fg):
            b = i_mod6 % NBUF
            q = i_mod6 % NI
            gather_wait(b, q)
            scale(b, q)
            scatter_start(b, q)
            if do_d:
                b1 = (i_mod6 + NBUF - 1) % NBUF
                q1 = (i_mod6 + NI - 1) % NI
                scatter_wait(b1, q1)
            if do_e:
                idx_load(i_dyn + 4, (i_mod6 + 4) % NI)
            if do_fg:
                q2 = (i_mod6 + 2) % NI
                b2 = (i_mod6 + 2) % NBUF
                idx_wait(i_dyn + 2, q2)
                gather_start(b2, q2)

        step(row0 + 0, 0, False, True, True)
        step(row0 + 1, 1, True, True, True)

        def super_body(k, _):
            base = row0 + 2 + 6 * k
            for j in range(6):
                step(base + j, 2 + j, True, True, True)
            return 0

        lax.fori_loop(0, rpt // 6 - 1, super_body, 0)

        tbase = row0 + rpt - 4
        step(tbase + 0, 2, True, False, True)
        step(tbase + 1, 3, True, False, True)
        step(tbase + 2, 4, True, False, False)
        step(tbase + 3, 5, True, False, False)
        scatter_wait(5 % NBUF, 5 % NI)

        plsc.subcore_barrier()

        off = 0
        while off < NODES_PER_TILE:
            n = min(CHUNK, NODES_PER_TILE - off)
            nrow = nbase + off
            pltpu.sync_copy(acc.at[pl.ds(nrow, n)], rows[0].at[pl.ds(0, n)])
            pltpu.sync_copy(rows[0].at[pl.ds(0, n)],
                            out_hbm.at[cid, pl.ds(nrow, n)])
            off += n

    return sc_spmm


_sc_spmm = _make_sc_spmm()


def _finish_body(p0_ref, p1_ref, x_ref, a_ref, x0_ref, o_ref):
    ax = p0_ref[0] + p1_ref[0]
    alph = jax.nn.sigmoid(a_ref[...])
    o_ref[...] = alph * (0.5 * (ax - x_ref[...])) + x0_ref[...]


def _finish(partial, x, alpha, x0):
    blk = 1000
    grid = N_NODES // blk
    return pl.pallas_call(
        _finish_body,
        grid=(grid,),
        in_specs=[
            pl.BlockSpec((1, blk, D), lambda i: (0, i, 0)),
            pl.BlockSpec((1, blk, D), lambda i: (1, i, 0)),
            pl.BlockSpec((blk, D), lambda i: (i, 0)),
            pl.BlockSpec((blk, 1), lambda i: (i, 0)),
            pl.BlockSpec((blk, D), lambda i: (i, 0)),
        ],
        out_specs=pl.BlockSpec((blk, D), lambda i: (i, 0)),
        out_shape=jax.ShapeDtypeStruct((N_NODES, D), jnp.float32),
    )(partial, partial, x, alpha, x0)


@jax.jit
def kernel(t, x, edge_index, adj_values, alpha_train, x0):
    del t
    src = edge_index[1].astype(jnp.int32)
    dst = edge_index[0].astype(jnp.int32)
    vals = adj_values.astype(jnp.float32)

    t0 = TAIL_CHUNK0 * CHUNK
    pad = TAIL_LEN - (N_EDGES - t0)
    tsrc = jnp.pad(src[t0:], (0, pad))
    tdst = jnp.pad(dst[t0:], (0, pad))
    tvals = jnp.pad(vals[t0:], (0, pad))

    partial = _sc_spmm(x, src, dst, vals, tsrc, tdst, tvals)
    return _finish(partial, x, alpha_train.reshape(-1, 1), x0)


# speedup vs baseline: 1.0736x; 1.0048x over previous
import functools

import jax
import jax.numpy as jnp
from jax import lax
from jax.experimental import pallas as pl
from jax.experimental.pallas import tpu as pltpu
from jax.experimental.pallas import tpu_sc as plsc

N_NODES = 10000
N_EDGES = 320000
D = 128

NC = 2
NS = 16
NW = NC * NS

CHUNK = 112
NROWS = 2880
E_PAD = NROWS * CHUNK
TAIL_CHUNK0 = N_EDGES // CHUNK
TAIL_CHUNK0 = (N_EDGES // CHUNK // 8) * 8
TAIL_LEN = (NROWS - TAIL_CHUNK0) * CHUNK
RPT0 = 138
RPT1 = (NROWS - NS * RPT0) // NS
N_PAD = 10112
NODES_PER_TILE = N_PAD // NS
NBUF = 3
NI = 6
GROUPS = CHUNK // 16


def _make_sc_spmm():
    mesh = plsc.VectorSubcoreMesh(core_axis_name="c", subcore_axis_name="s")

    @functools.partial(
        pl.kernel,
        out_type=jax.ShapeDtypeStruct((NC, N_PAD, D), jnp.float32),
        mesh=mesh,
        scratch_types=[
            pltpu.VMEM((NI, CHUNK), jnp.int32),
            pltpu.VMEM((NI, CHUNK), jnp.int32),
            pltpu.VMEM((NI, CHUNK), jnp.float32),
            [pltpu.VMEM((CHUNK, D), jnp.float32)] * NBUF,
            pltpu.VMEM_SHARED((N_PAD, D), jnp.float32),
            [pltpu.SemaphoreType.DMA] * NI,
            [pltpu.SemaphoreType.DMA] * NBUF,
            [pltpu.SemaphoreType.DMA] * NBUF,
        ],
    )
    def sc_spmm(x_hbm, src_hbm, dst_hbm, vals_hbm,
                tsrc_hbm, tdst_hbm, tvals_hbm, out_hbm,
                src_s, dst_s, val_s, rows, acc, isem, gsem, ssem):
        cid = lax.axis_index("c")
        sid = lax.axis_index("s")
        rpt = jnp.where(cid == 0, RPT0, RPT1)
        row0 = jnp.where(cid == 0, sid * RPT0, NS * RPT0 + sid * RPT1)

        zeros16 = jnp.zeros((16,), jnp.float32)
        dn = lax.GatherDimensionNumbers(
            offset_dims=(), collapsed_slice_dims=(0,), start_index_map=(0,))

        def idx_load(c, q):
            @pl.when(c < TAIL_CHUNK0)
            def _():
                off = c * CHUNK
                pltpu.async_copy(src_hbm.at[pl.ds(off, CHUNK)],
                                 src_s.at[q], isem[q])
                pltpu.async_copy(dst_hbm.at[pl.ds(off, CHUNK)],
                                 dst_s.at[q], isem[q])
                pltpu.async_copy(vals_hbm.at[pl.ds(off, CHUNK)],
                                 val_s.at[q], isem[q])

            @pl.when(c >= TAIL_CHUNK0)
            def _():
                off = (c - TAIL_CHUNK0) * CHUNK
                pltpu.async_copy(tsrc_hbm.at[pl.ds(off, CHUNK)],
                                 src_s.at[q], isem[q])
                pltpu.async_copy(tdst_hbm.at[pl.ds(off, CHUNK)],
                                 dst_s.at[q], isem[q])
                pltpu.async_copy(tvals_hbm.at[pl.ds(off, CHUNK)],
                                 val_s.at[q], isem[q])

        def idx_wait(c, q):
            for ref, slot in ((src_hbm, src_s), (dst_hbm, dst_s),
                              (vals_hbm, val_s)):
                pltpu.make_async_copy(ref.at[pl.ds(0, CHUNK)],
                                      slot.at[q], isem[q]).wait()


        for q in range(4):
            idx_load(row0 + q, q)

        def zrow(i, _):
            for d in range(D // 16):
                rows[0][i, pl.ds(d * 16, 16)] = zeros16
            return 0

        lax.fori_loop(0, CHUNK, zrow, 0)
        nbase = sid * NODES_PER_TILE
        off = 0
        while off < NODES_PER_TILE:
            n = min(CHUNK, NODES_PER_TILE - off)
            pltpu.sync_copy(rows[0].at[pl.ds(0, n)],
                            acc.at[pl.ds(nbase + off, n)])
            off += n
        plsc.subcore_barrier()

        def gather_start(b, q):
            pltpu.async_copy(x_hbm.at[src_s.at[q]], rows[b], gsem[b])

        def gather_wait(b, q):
            pltpu.make_async_copy(x_hbm.at[src_s.at[q]], rows[b],
                                  gsem[b]).wait()

        def scatter_start(b, q):
            pltpu.async_copy(rows[b], acc.at[dst_s.at[q]], ssem[b], add=True)

        def scatter_wait(b, q):
            pltpu.make_async_copy(rows[b], acc.at[dst_s.at[q]],
                                  ssem[b]).wait()

        def scale(b, q):
            def group_body(g, _):
                v = val_s[q, pl.ds(g * 16, 16)]
                for j in range(16):
                    s = lax.gather(v, jnp.full((16, 1), j, jnp.int32), dn,
                                   (1,),
                                   mode=lax.GatherScatterMode.PROMISE_IN_BOUNDS)
                    e = g * 16 + j
                    for d in range(D // 16):
                        seg = rows[b][e, pl.ds(d * 16, 16)]
                        rows[b][e, pl.ds(d * 16, 16)] = seg * s
                return 0

            lax.fori_loop(0, GROUPS, group_body, 0)

        idx_wait(row0 + 0, 0)
        gather_start(0, 0)
        idx_wait(row0 + 1, 1)
        gather_start(1, 1)

        def step(i_dyn, i_mod6, do_d, do_e, do_fg):
            b = i_mod6 % NBUF
            q = i_mod6 % NI
            gather_wait(b, q)
            scale(b, q)
            scatter_start(b, q)
            if do_d:
                b1 = (i_mod6 + NBUF - 1) % NBUF
                q1 = (i_mod6 + NI - 1) % NI
                scatter_wait(b1, q1)
            if do_e:
                idx_load(i_dyn + 4, (i_mod6 + 4) % NI)
            if do_fg:
                q2 = (i_mod6 + 2) % NI
                b2 = (i_mod6 + 2) % NBUF
                idx_wait(i_dyn + 2, q2)
                gather_start(b2, q2)

        step(row0 + 0, 0, False, True, True)
        step(row0 + 1, 1, True, True, True)

        def super_body(k, _):
            base = row0 + 2 + 6 * k
            for j in range(6):
                step(base + j, 2 + j, True, True, True)
            return 0

        lax.fori_loop(0, rpt // 6 - 1, super_body, 0)

        tbase = row0 + rpt - 4
        step(tbase + 0, 2, True, False, True)
        step(tbase + 1, 3, True, False, True)
        step(tbase + 2, 4, True, False, False)
        step(tbase + 3, 5, True, False, False)
        scatter_wait(5 % NBUF, 5 % NI)

        plsc.subcore_barrier()

        pltpu.sync_copy(acc.at[pl.ds(nbase, NODES_PER_TILE)],
                        out_hbm.at[cid, pl.ds(nbase, NODES_PER_TILE)])

    return sc_spmm


_sc_spmm = _make_sc_spmm()


def _finish_body(p0_ref, p1_ref, x_ref, a_ref, x0_ref, o_ref):
    ax = p0_ref[0] + p1_ref[0]
    alph = jax.nn.sigmoid(a_ref[...])
    o_ref[...] = alph * (0.5 * (ax - x_ref[...])) + x0_ref[...]


def _finish(partial, x, alpha, x0):
    blk = 1000
    grid = N_NODES // blk
    return pl.pallas_call(
        _finish_body,
        grid=(grid,),
        in_specs=[
            pl.BlockSpec((1, blk, D), lambda i: (0, i, 0)),
            pl.BlockSpec((1, blk, D), lambda i: (1, i, 0)),
            pl.BlockSpec((blk, D), lambda i: (i, 0)),
            pl.BlockSpec((blk, 1), lambda i: (i, 0)),
            pl.BlockSpec((blk, D), lambda i: (i, 0)),
        ],
        out_specs=pl.BlockSpec((blk, D), lambda i: (i, 0)),
        out_shape=jax.ShapeDtypeStruct((N_NODES, D), jnp.float32),
    )(partial, partial, x, alpha, x0)


@jax.jit
def kernel(t, x, edge_index, adj_values, alpha_train, x0):
    del t
    src = edge_index[1].astype(jnp.int32)
    dst = edge_index[0].astype(jnp.int32)
    vals = adj_values.astype(jnp.float32)

    t0 = TAIL_CHUNK0 * CHUNK
    pad = TAIL_LEN - (N_EDGES - t0)
    tsrc = jnp.pad(src[t0:], (0, pad))
    tdst = jnp.pad(dst[t0:], (0, pad))
    tvals = jnp.pad(vals[t0:], (0, pad))

    partial = _sc_spmm(x, src, dst, vals, tsrc, tdst, tvals)
    return _finish(partial, x, alpha_train.reshape(-1, 1), x0)
